# Initial kernel scaffold; baseline (speedup 1.0000x reference)
import jax
import jax.numpy as jnp
from jax.experimental import pallas as pl


def kernel(x_input, edge_index, W1, b1, W2, b2, Wadj, badj, Wd1, bd1, Wd2, bd2, Wd3, bd3, Wm, bm, Ws, bs, Wp, bp, mu):
    raise NotImplementedError("write your pallas kernel here")


# speedup vs baseline: 3.8785x; 3.8785x over previous
import functools

import jax
import jax.numpy as jnp
from jax import lax
from jax.experimental import pallas as pl
from jax.experimental.pallas import tpu as pltpu
from jax.experimental.pallas import tpu_sc as plsc

N = 2048
IN_DIM = 3000
E = 32768
HID = 128
LAT = 15
NC = 10

_F32 = jnp.float32
_HI = lax.Precision.HIGHEST

_NSUB = 16
_EPS = E // _NSUB
_CHUNK = 128
_NCHUNK = _EPS // _CHUNK
_ROWS = 256
_ACC_REAL = _ROWS * N
_ACC_WORDS = _ACC_REAL + N
_ZCH = _ACC_WORDS // _NSUB
_WB = _ACC_REAL // _NSUB
_NPASS = N // (2 * _ROWS)


def _build_adjacency(dst, src):
  mesh = plsc.VectorSubcoreMesh(core_axis_name="c", subcore_axis_name="s",
                                num_cores=2, num_subcores=_NSUB)

  @functools.partial(
      pl.kernel,
      out_type=jax.ShapeDtypeStruct((N * N,), _F32),
      mesh=mesh,
      scratch_types=[
          pltpu.VMEM((_EPS,), jnp.int32),
          pltpu.VMEM((_EPS,), jnp.int32),
          pltpu.VMEM((_NCHUNK, _CHUNK), jnp.int32),
          pltpu.VMEM((_CHUNK,), _F32),
          pltpu.VMEM((_ZCH,), _F32),
          pltpu.VMEM_SHARED((_ACC_WORDS,), _F32),
          pltpu.SemaphoreType.DMA,
      ],
  )
  def build(dst_hbm, src_hbm, out_hbm, dst_v, src_v, idx_v, ones_v, zero_v,
            acc, sem):
    core = lax.axis_index("c")
    sub = lax.axis_index("s")
    ebase = sub * _EPS
    pltpu.sync_copy(dst_hbm.at[pl.ds(ebase, _EPS)], dst_v)
    pltpu.sync_copy(src_hbm.at[pl.ds(ebase, _EPS)], src_v)

    def _zinit(i, carry):
      zero_v[pl.ds(i * 16, 16)] = jnp.zeros((16,), _F32)
      return carry

    lax.fori_loop(0, _ZCH // 16, _zinit, 0)
    for i in range(_CHUNK // 16):
      ones_v[pl.ds(i * 16, 16)] = jnp.ones((16,), _F32)

    for p in range(_NPASS):
      row_base = (core * _NPASS + p) * _ROWS
      pltpu.sync_copy(zero_v, acc.at[pl.ds(sub * _ZCH, _ZCH)])
      plsc.subcore_barrier()

      copies = []
      for j in range(_NCHUNK):
        def _mkidx(i, carry, j=j):
          off = j * _CHUNK + i * 16
          d = dst_v[pl.ds(off, 16)]
          s = src_v[pl.ds(off, 16)]
          loc = d - row_base
          ok = (loc >= 0) & (loc < _ROWS)
          idx_v[j, pl.ds(i * 16, 16)] = jnp.where(ok, loc * N + s,
                                                  _ACC_REAL + s)
          return carry

        lax.fori_loop(0, _CHUNK // 16, _mkidx, 0)
        copies.append(
            pltpu.async_copy(ones_v, acc.at[idx_v.at[j]], sem, add=True))
      for c in copies:
        c.wait()
      plsc.subcore_barrier()

      pltpu.sync_copy(
          acc.at[pl.ds(sub * _WB, _WB)],
          out_hbm.at[pl.ds((row_base + sub * (_ROWS // _NSUB)) * N, _WB)])
      plsc.subcore_barrier()

  return build(dst, src)


_RB = 256
_GRID = N // _RB


def _proj_kernel(x_ref, w_ref, o_ref):
  o_ref[...] = jnp.dot(x_ref[...], w_ref[...],
                       preferred_element_type=_F32, precision=_HI)


def _project(x, w1):
  return pl.pallas_call(
      _proj_kernel,
      grid=(_GRID, 4),
      in_specs=[
          pl.BlockSpec((_RB, IN_DIM), lambda i, j: (i, 0)),
          pl.BlockSpec((IN_DIM, HID), lambda i, j: (j, 0)),
      ],
      out_specs=pl.BlockSpec((_RB, HID), lambda i, j: (i, j)),
      out_shape=jax.ShapeDtypeStruct((N, 4 * HID), _F32),
  )(x, w1)


def _hops_kernel(a_ref, p_ref, b1_ref, w2_ref, b2_ref, z_o, nrm_s, ta, tb):
  h = pl.program_id(0)

  def mm(x, w):
    return jnp.dot(x, w, preferred_element_type=_F32, precision=_HI)

  @pl.when(h == 0)
  def _():
    for i in range(_GRID):
      blk = a_ref[pl.ds(i * _RB, _RB), :]
      deg = jnp.sum(blk, axis=1, keepdims=True)
      nrm_s[pl.ds(i * _RB, _RB), :] = lax.rsqrt(jnp.maximum(deg, 1.0))

  plan = [(1, None, ta, None), (2, ta, tb, None), (3, tb, ta, 0),
          (4, ta, tb, 1), (5, tb, ta, 2), (6, ta, tb, 3)]
  for step, src, dst, zw in plan:
    @pl.when(h == step)
    def _(step=step, src=src, dst=dst, zw=zw):
      if src is None:
        tin = p_ref[:, 384:512]
      else:
        tin = src[...]
      sin = nrm_s[...] * tin
      for i in range(_GRID):
        rows = pl.ds(i * _RB, _RB)
        blk = nrm_s[rows, :] * mm(a_ref[rows, :], sin)
        if step == 1:
          blk = blk + p_ref[rows, 256:384]
        elif step == 2:
          blk = blk + p_ref[rows, 128:256]
        elif step == 3:
          blk = blk + p_ref[rows, 0:128] + b1_ref[...]
        dst[rows, :] = blk
      if zw is not None:
        zpart = mm(dst[...], w2_ref[pl.ds(zw * HID, HID), :])
        if zw == 0:
          z_o[...] = zpart + b2_ref[...]
        else:
          z_o[...] = z_o[...] + zpart


def _hops(a, p, b1, w2, b2):
  return pl.pallas_call(
      _hops_kernel,
      grid=(7,),
      in_specs=[
          pl.BlockSpec((N, N), lambda h: (0, 0)),
          pl.BlockSpec((N, 4 * HID), lambda h: (0, 0)),
          pl.BlockSpec((1, HID), lambda h: (0, 0)),
          pl.BlockSpec((4 * HID, LAT), lambda h: (0, 0)),
          pl.BlockSpec((1, LAT), lambda h: (0, 0)),
      ],
      out_specs=pl.BlockSpec((N, LAT), lambda h: (0, 0)),
      out_shape=jax.ShapeDtypeStruct((N, LAT), _F32),
      scratch_shapes=[
          pltpu.VMEM((N, 1), _F32),
          pltpu.VMEM((N, HID), _F32),
          pltpu.VMEM((N, HID), _F32),
      ],
  )(a, p, b1, w2, b2)


def _post_kernel(z_ref, wa_ref, ba_ref, wd1_ref, bd1_ref, wd2_ref, bd2_ref,
                 wd3_ref, bd3_ref, mu_ref, q_o, h3_o, f1_o, f2_o):
  def mm(x, w):
    return jnp.dot(x, w, preferred_element_type=_F32, precision=_HI)

  z = z_ref[...]

  m = mu_ref[...]
  zm = lax.dot_general(z, m, (((1,), (1,)), ((), ())),
                       preferred_element_type=_F32, precision=_HI)
  mu2 = lax.dot_general(jnp.ones((1, LAT), _F32), m * m,
                        (((1,), (1,)), ((), ())),
                        preferred_element_type=_F32, precision=_HI)
  d2 = jnp.sum(z * z, axis=1, keepdims=True) - 2.0 * zm + mu2
  qn = 1.0 / (1.0 + d2)
  q_o[...] = qn / jnp.sum(qn, axis=1, keepdims=True)

  h1 = jnp.maximum(mm(z, wd1_ref[...]) + bd1_ref[...], 0.0)
  h2 = jnp.maximum(mm(h1, wd2_ref[...]) + bd2_ref[...], 0.0)
  h3 = jnp.maximum(mm(h2, wd3_ref[...]) + bd3_ref[...], 0.0)
  h3_o[...] = h3

  wa = wa_ref[...]
  ba = ba_ref[...]
  cc = lax.dot_general(wa, wa, (((1,), (1,)), ((), ())),
                       preferred_element_type=_F32, precision=_HI)
  u = lax.dot_general(wa, ba, (((1,), (1,)), ((), ())),
                      preferred_element_type=_F32, precision=_HI)
  s = lax.dot_general(ba, ba, (((1,), (1,)), ((), ())),
                      preferred_element_type=_F32, precision=_HI)
  zc = mm(z, cc)
  zu = mm(z, u)
  one = jnp.ones((N, 1), _F32)
  f1_o[...] = jnp.concatenate([zc, one, zu + s], axis=1)
  f2_o[...] = jnp.concatenate([z, zu, one], axis=1)


def _post(z, wa, ba, wd1, bd1, wd2, bd2, wd3, bd3, mu):
  return pl.pallas_call(
      _post_kernel,
      out_shape=[
          jax.ShapeDtypeStruct((N, NC), _F32),
          jax.ShapeDtypeStruct((N, 512), _F32),
          jax.ShapeDtypeStruct((N, LAT + 2), _F32),
          jax.ShapeDtypeStruct((N, LAT + 2), _F32),
      ],
  )(z, wa, ba, wd1, bd1, wd2, bd2, wd3, bd3, mu)


def _adj_kernel(f1_ref, f2_ref, o_ref):
  g = lax.dot_general(f1_ref[...], f2_ref[...], (((1,), (1,)), ((), ())),
                      preferred_element_type=_F32, precision=_HI)
  o_ref[...] = 1.0 / (1.0 + jnp.exp(-g))


def _adj_decoder(f1, f2):
  return pl.pallas_call(
      _adj_kernel,
      grid=(_GRID,),
      in_specs=[
          pl.BlockSpec((_RB, LAT + 2), lambda i: (i, 0)),
          pl.BlockSpec((N, LAT + 2), lambda i: (0, 0)),
      ],
      out_specs=pl.BlockSpec((_RB, N), lambda i: (i, 0)),
      out_shape=jax.ShapeDtypeStruct((N, N), _F32),
  )(f1, f2)


def _head(h3, w, b, act):
  def body(h_ref, w_ref, b_ref, o_ref):
    o_ref[...] = act(
        jnp.dot(h_ref[...], w_ref[...], preferred_element_type=_F32,
                precision=_HI) + b_ref[...])

  return pl.pallas_call(
      body,
      grid=(_GRID,),
      in_specs=[
          pl.BlockSpec((_RB, 512), lambda i: (i, 0)),
          pl.BlockSpec((512, IN_DIM), lambda i: (0, 0)),
          pl.BlockSpec((1, IN_DIM), lambda i: (0, 0)),
      ],
      out_specs=pl.BlockSpec((_RB, IN_DIM), lambda i: (i, 0)),
      out_shape=jax.ShapeDtypeStruct((N, IN_DIM), _F32),
  )(h3, w, b)


def _act_mean(y):
  return jnp.clip(jnp.exp(y), 1e-5, 1e6)


def _act_disp(y):
  sp = jnp.maximum(y, 0.0) + jnp.log(1.0 + jnp.exp(-jnp.abs(y)))
  return jnp.clip(sp, 1e-4, 1e4)


def _act_pi(y):
  return 1.0 / (1.0 + jnp.exp(-y))


def kernel(x_input, edge_index, W1, b1, W2, b2, Wadj, badj, Wd1, bd1, Wd2,
           bd2, Wd3, bd3, Wm, bm, Ws, bs, Wp, bp, mu):
  src = edge_index[0]
  dst = edge_index[1]

  a = _build_adjacency(dst, src).reshape(N, N)
  p = _project(x_input, W1)
  z = _hops(a, p, b1.reshape(1, HID), W2, b2.reshape(1, LAT))
  q, h3, f1, f2 = _post(
      z, Wadj, badj.reshape(1, N), Wd1, bd1.reshape(1, 128), Wd2,
      bd2.reshape(1, 256), Wd3, bd3.reshape(1, 512), mu)

  adj_out = _adj_decoder(f1, f2)
  _mean = _head(h3, Wm, bm.reshape(1, IN_DIM), _act_mean)
  _disp = _head(h3, Ws, bs.reshape(1, IN_DIM), _act_disp)
  _pi = _head(h3, Wp, bp.reshape(1, IN_DIM), _act_pi)
  return (adj_out, z, q, _mean, _disp, _pi)


# speedup vs baseline: 20.4438x; 5.2711x over previous
import functools

import jax
import jax.numpy as jnp
from jax import lax
from jax.experimental import pallas as pl
from jax.experimental.pallas import tpu as pltpu
from jax.experimental.pallas import tpu_sc as plsc

N = 2048
IN_DIM = 3000
E = 32768
HID = 128
LAT = 15
NC = 10

_F32 = jnp.float32
_HI = lax.Precision.HIGHEST

_NSUB = 16
_EPS = E // _NSUB
_CHUNK = 128
_NCHUNK = _EPS // _CHUNK
_ROWS = 256
_ACC_REAL = _ROWS * N
_ACC_WORDS = _ACC_REAL + 2 * N
_ZCH = _ACC_WORDS // _NSUB
_WB = _ACC_REAL // _NSUB
_NPASS = N // (2 * _ROWS)


def _build_adjacency(dst, src):
  mesh = plsc.VectorSubcoreMesh(core_axis_name="c", subcore_axis_name="s",
                                num_cores=2, num_subcores=_NSUB)

  @functools.partial(
      pl.kernel,
      out_type=jax.ShapeDtypeStruct((N * N,), _F32),
      mesh=mesh,
      scratch_types=[
          pltpu.VMEM((_EPS,), jnp.int32),
          pltpu.VMEM((_EPS,), jnp.int32),
          pltpu.VMEM((_NCHUNK, _CHUNK), jnp.int32),
          pltpu.VMEM((_CHUNK,), _F32),
          pltpu.VMEM((_ZCH,), _F32),
          pltpu.VMEM_SHARED((_ACC_WORDS,), _F32),
          pltpu.SemaphoreType.DMA,
      ],
  )
  def build(dst_hbm, src_hbm, out_hbm, dst_v, src_v, idx_v, ones_v, zero_v,
            acc, sem):
    core = lax.axis_index("c")
    sub = lax.axis_index("s")
    ebase = sub * _EPS
    pltpu.sync_copy(dst_hbm.at[pl.ds(ebase, _EPS)], dst_v)
    pltpu.sync_copy(src_hbm.at[pl.ds(ebase, _EPS)], src_v)

    def _zinit(i, carry):
      zero_v[pl.ds(i * 16, 16)] = jnp.zeros((16,), _F32)
      return carry

    lax.fori_loop(0, _ZCH // 16, _zinit, 0)
    for i in range(_CHUNK // 16):
      ones_v[pl.ds(i * 16, 16)] = jnp.ones((16,), _F32)

    for p in range(_NPASS):
      row_base = (core * _NPASS + p) * _ROWS
      pltpu.sync_copy(zero_v, acc.at[pl.ds(sub * _ZCH, _ZCH)])
      plsc.subcore_barrier()

      copies = []
      for j in range(_NCHUNK):
        def _mkidx(i, carry, j=j):
          off = j * _CHUNK + i * 16
          d = dst_v[pl.ds(off, 16)]
          s = src_v[pl.ds(off, 16)]
          loc = d - row_base
          ok = (loc >= 0) & (loc < _ROWS)
          idx_v[j, pl.ds(i * 16, 16)] = jnp.where(ok, loc * N + s,
                                                  _ACC_REAL + s)
          return carry

        lax.fori_loop(0, _CHUNK // 16, _mkidx, 0)
        copies.append(
            pltpu.async_copy(ones_v, acc.at[idx_v.at[j]], sem, add=True))
      for c in copies:
        c.wait()
      plsc.subcore_barrier()

      pltpu.sync_copy(
          acc.at[pl.ds(sub * _WB, _WB)],
          out_hbm.at[pl.ds((row_base + sub * (_ROWS // _NSUB)) * N, _WB)])
      plsc.subcore_barrier()

  return build(dst, src)


_RB = 256
_GRID = N // _RB


def _proj_kernel(xt_ref, w_ref, o_ref):
  dn = (((0,), (0,)), ((), ()))
  xt = xt_ref[...]
  for j in range(4):
    o_ref[:, pl.ds(j * HID, HID)] = lax.dot_general(
        xt, w_ref[j, :, :], dn, preferred_element_type=_F32)


def _project(xt, w1):
  return pl.pallas_call(
      _proj_kernel,
      grid=(_GRID,),
      in_specs=[
          pl.BlockSpec((IN_DIM, _RB), lambda i: (0, i)),
          pl.BlockSpec((4, IN_DIM, HID), lambda i: (0, 0, 0)),
      ],
      out_specs=pl.BlockSpec((_RB, 4 * HID), lambda i: (i, 0)),
      out_shape=jax.ShapeDtypeStruct((N, 4 * HID), _F32),
  )(xt, w1.reshape(4, IN_DIM, HID))


def _hops_kernel(a_ref, p_ref, b1_ref, w2_ref, b2_ref, z_o, nrm_s, ta, tb):
  h = pl.program_id(0)

  def mm(x, w):
    return jnp.dot(x, w, preferred_element_type=_F32)

  @pl.when(h == 0)
  def _():
    for i in range(_GRID):
      blk = a_ref[pl.ds(i * _RB, _RB), :].astype(_F32)
      deg = jnp.sum(blk, axis=1, keepdims=True)
      nrm_s[pl.ds(i * _RB, _RB), :] = lax.rsqrt(jnp.maximum(deg, 1.0))

  plan = [(1, None, ta, None), (2, ta, tb, None), (3, tb, ta, 0),
          (4, ta, tb, 1), (5, tb, ta, 2), (6, ta, tb, 3)]
  for step, src, dst, zw in plan:
    @pl.when(h == step)
    def _(step=step, src=src, dst=dst, zw=zw):
      if src is None:
        tin = p_ref[:, 384:512]
      else:
        tin = src[...]
      hi = (nrm_s[...] * tin).astype(jnp.bfloat16)
      for i in range(_GRID):
        rows = pl.ds(i * _RB, _RB)
        acc = jnp.dot(a_ref[rows, :], hi, preferred_element_type=_F32)
        blk = nrm_s[rows, :] * acc
        if step == 1:
          blk = blk + p_ref[rows, 256:384]
        elif step == 2:
          blk = blk + p_ref[rows, 128:256]
        elif step == 3:
          blk = blk + p_ref[rows, 0:128] + b1_ref[...]
        dst[rows, :] = blk
      if zw is not None:
        zpart = mm(dst[...], w2_ref[pl.ds(zw * HID, HID), :])
        if zw == 0:
          z_o[...] = zpart + b2_ref[...]
        else:
          z_o[...] = z_o[...] + zpart


def _hops(a, p, b1, w2, b2):
  return pl.pallas_call(
      _hops_kernel,
      grid=(7,),
      in_specs=[
          pl.BlockSpec((N, N), lambda h: (0, 0)),
          pl.BlockSpec((N, 4 * HID), lambda h: (0, 0)),
          pl.BlockSpec((1, HID), lambda h: (0, 0)),
          pl.BlockSpec((4 * HID, LAT), lambda h: (0, 0)),
          pl.BlockSpec((1, LAT), lambda h: (0, 0)),
      ],
      out_specs=pl.BlockSpec((N, LAT), lambda h: (0, 0)),
      out_shape=jax.ShapeDtypeStruct((N, LAT), _F32),
      scratch_shapes=[
          pltpu.VMEM((N, 1), _F32),
          pltpu.VMEM((N, HID), _F32),
          pltpu.VMEM((N, HID), _F32),
      ],
  )(a, p, b1, w2, b2)


def _post_kernel(z_ref, wa_ref, ba_ref, wd1_ref, bd1_ref, wd2_ref, bd2_ref,
                 wd3_ref, bd3_ref, mu_ref, q_o, h3_o, f1_o, f2_o):
  def mm(x, w):
    return jnp.dot(x, w, preferred_element_type=_F32, precision=_HI)

  z = z_ref[...]

  m = mu_ref[...]
  zm = lax.dot_general(z, m, (((1,), (1,)), ((), ())),
                       preferred_element_type=_F32, precision=_HI)
  mu2 = lax.dot_general(jnp.ones((1, LAT), _F32), m * m,
                        (((1,), (1,)), ((), ())),
                        preferred_element_type=_F32, precision=_HI)
  d2 = jnp.sum(z * z, axis=1, keepdims=True) - 2.0 * zm + mu2
  qn = 1.0 / (1.0 + d2)
  q_o[...] = qn / jnp.sum(qn, axis=1, keepdims=True)

  def md(x, w):
    return jnp.dot(x, w, preferred_element_type=_F32)

  h1 = jnp.maximum(md(z, wd1_ref[...]) + bd1_ref[...], 0.0)
  h2 = jnp.maximum(md(h1, wd2_ref[...]) + bd2_ref[...], 0.0)
  h3 = jnp.maximum(md(h2, wd3_ref[...]) + bd3_ref[...], 0.0)
  h3_o[...] = h3

  wa = wa_ref[...]
  ba = ba_ref[...]
  cc = lax.dot_general(wa, wa, (((1,), (1,)), ((), ())),
                       preferred_element_type=_F32, precision=_HI)
  u = lax.dot_general(wa, ba, (((1,), (1,)), ((), ())),
                      preferred_element_type=_F32, precision=_HI)
  s = lax.dot_general(ba, ba, (((1,), (1,)), ((), ())),
                      preferred_element_type=_F32, precision=_HI)
  zc = mm(z, cc)
  zu = mm(z, u)
  one = jnp.ones((N, 1), _F32)
  f1_o[...] = jnp.concatenate([zc, one, zu + s], axis=1)
  f2_o[...] = jnp.concatenate([z, zu, one], axis=1)


def _post(z, wa, ba, wd1, bd1, wd2, bd2, wd3, bd3, mu):
  return pl.pallas_call(
      _post_kernel,
      out_shape=[
          jax.ShapeDtypeStruct((N, NC), _F32),
          jax.ShapeDtypeStruct((N, 512), _F32),
          jax.ShapeDtypeStruct((N, LAT + 2), _F32),
          jax.ShapeDtypeStruct((N, LAT + 2), _F32),
      ],
  )(z, wa, ba, wd1, bd1, wd2, bd2, wd3, bd3, mu)


def _adj_kernel(f1_ref, f2_ref, o_ref):
  g = lax.dot_general(f1_ref[...], f2_ref[...], (((1,), (1,)), ((), ())),
                      preferred_element_type=_F32, precision=_HI)
  o_ref[...] = 1.0 / (1.0 + jnp.exp(-g))


def _adj_decoder(f1, f2):
  return pl.pallas_call(
      _adj_kernel,
      grid=(_GRID,),
      in_specs=[
          pl.BlockSpec((_RB, LAT + 2), lambda i: (i, 0)),
          pl.BlockSpec((N, LAT + 2), lambda i: (0, 0)),
      ],
      out_specs=pl.BlockSpec((_RB, N), lambda i: (i, 0)),
      out_shape=jax.ShapeDtypeStruct((N, N), _F32),
  )(f1, f2)


def _head(h3, w, b, act):
  def body(h_ref, wt_ref, b_ref, o_ref):
    g = lax.dot_general(wt_ref[...], h_ref[...], (((1,), (1,)), ((), ())),
                        preferred_element_type=_F32)
    o_ref[...] = act(g + b_ref[...])

  return pl.pallas_call(
      body,
      grid=(_GRID,),
      in_specs=[
          pl.BlockSpec((_RB, 512), lambda i: (i, 0)),
          pl.BlockSpec((IN_DIM, 512), lambda i: (0, 0)),
          pl.BlockSpec((IN_DIM, 1), lambda i: (0, 0)),
      ],
      out_specs=pl.BlockSpec((IN_DIM, _RB), lambda i: (0, i)),
      out_shape=jax.ShapeDtypeStruct((IN_DIM, N), _F32),
  )(h3, w.T, b)


def _act_mean(y):
  return jnp.clip(jnp.exp(y), 1e-5, 1e6)


def _act_disp(y):
  sp = jnp.maximum(y, 0.0) + jnp.log(1.0 + jnp.exp(-jnp.abs(y)))
  return jnp.clip(sp, 1e-4, 1e4)


def _act_pi(y):
  return 1.0 / (1.0 + jnp.exp(-y))


def kernel(x_input, edge_index, W1, b1, W2, b2, Wadj, badj, Wd1, bd1, Wd2,
           bd2, Wd3, bd3, Wm, bm, Ws, bs, Wp, bp, mu):
  src = edge_index[0]
  dst = edge_index[1]

  a = _build_adjacency(dst, src).reshape(N, N).astype(jnp.bfloat16)
  p = _project(x_input.T, W1)
  z = _hops(a, p, b1.reshape(1, HID), W2, b2.reshape(1, LAT))
  q, h3, f1, f2 = _post(
      z, Wadj, badj.reshape(1, N), Wd1, bd1.reshape(1, 128), Wd2,
      bd2.reshape(1, 256), Wd3, bd3.reshape(1, 512), mu)

  adj_out = _adj_decoder(f1, f2)
  _mean = _head(h3, Wm, bm.reshape(IN_DIM, 1), _act_mean).T
  _disp = _head(h3, Ws, bs.reshape(IN_DIM, 1), _act_disp).T
  _pi = _head(h3, Wp, bp.reshape(IN_DIM, 1), _act_pi).T
  return (adj_out, z, q, _mean, _disp, _pi)
